# baseline (device time: 129605 ns/iter reference)
import jax
import jax.numpy as jnp
from jax import lax
from jax.experimental import pallas as pl
from jax.experimental.pallas import tpu as pltpu

N_DEV = 4
DH = 64
BLK = 64


def _ring_allreduce(partial):
    m, n = partial.shape

    def body(p_ref, out_ref, comm_ref, send_sems, recv_sems):
        my = lax.axis_index("i")
        left = lax.rem(my + N_DEV - 1, N_DEV)
        right = lax.rem(my + 1, N_DEV)

        barrier_sem = pltpu.get_barrier_semaphore()
        for nbr in (left, right):
            pl.semaphore_signal(
                barrier_sem, inc=1,
                device_id=(nbr,), device_id_type=pl.DeviceIdType.MESH,
            )
        pl.semaphore_wait(barrier_sem, 2)

        out_ref[:, :] = p_ref[:, :]
        comm_ref[0, :, :] = p_ref[:, :]

        for h in range(N_DEV - 1):
            s = h % 2
            r = (h + 1) % 2
            rdma = pltpu.make_async_remote_copy(
                src_ref=comm_ref.at[s],
                dst_ref=comm_ref.at[r],
                send_sem=send_sems.at[s],
                recv_sem=recv_sems.at[r],
                device_id=(right,),
                device_id_type=pl.DeviceIdType.MESH,
            )
            rdma.start()
            rdma.wait()
            out_ref[:, :] += comm_ref[r, :, :]

    return pl.pallas_call(
        body,
        out_shape=jax.ShapeDtypeStruct((m, n), partial.dtype),
        in_specs=[pl.BlockSpec(memory_space=pltpu.VMEM)],
        out_specs=pl.BlockSpec(memory_space=pltpu.VMEM),
        scratch_shapes=[
            pltpu.VMEM((2, m, n), partial.dtype),
            pltpu.SemaphoreType.DMA((2,)),
            pltpu.SemaphoreType.DMA((2,)),
        ],
        compiler_params=pltpu.CompilerParams(collective_id=0),
    )(partial)


def kernel(x, Wq, K_ext, V_ext, Wo):
    B, Sq, E = x.shape
    hq = Wq.shape[1] // DH
    Skv = K_ext.shape[1]
    my = lax.axis_index("i")

    K = lax.dynamic_slice_in_dim(K_ext, my * hq, hq, axis=2)
    V = lax.dynamic_slice_in_dim(V_ext, my * hq, hq, axis=2)

    bf = jnp.bfloat16
    Q = (x.astype(bf) @ Wq.astype(bf)).reshape(B, Sq, hq, DH)

    qb = (jnp.arange(Sq) // BLK)[:, None]
    kb = (jnp.arange(Skv) // BLK)[None, :]
    mask = (qb == kb) | (kb == 0) | ((qb + kb) % 3 == 0)

    scores = jnp.einsum(
        "bihd,bjhd->bhij", Q, K.astype(bf),
        preferred_element_type=jnp.float32,
    ) * 0.125
    scores = jnp.where(mask[None, None, :, :], scores, -1e9)
    w = jax.nn.softmax(scores, axis=-1)

    ctx = jnp.einsum(
        "bhij,bjhd->bihd", w.astype(bf), V.astype(bf),
        preferred_element_type=jnp.float32,
    ).reshape(B, Sq, hq * DH)

    partial = jnp.einsum(
        "bse,eo->bso", ctx.astype(bf), Wo.astype(bf),
        preferred_element_type=jnp.float32,
    )

    out = _ring_allreduce(partial.reshape(B * Sq, E))
    return out.reshape(B, Sq, E)


# device time: 42045 ns/iter; 3.0825x vs baseline; 3.0825x over previous
import jax
import jax.numpy as jnp
from jax import lax
from jax.experimental import pallas as pl
from jax.experimental.pallas import tpu as pltpu

N_DEV = 4
DH = 64
BLK = 64


def _allreduce_rsag(partial_bf):
    m, n = partial_bf.shape
    c = m // N_DEV

    def body(p_ref, out_ref, send_buf, rs_buf, ag_buf, red_bf,
             rs_send_sems, rs_recv_sems, ag_send_sems, ag_recv_sems):
        my = lax.axis_index("i")

        for o in (1, 2, 3):
            peer = lax.rem(my + o, N_DEV)
            send_buf[o - 1, :, :] = p_ref[pl.ds(peer * c, c), :]

        barrier_sem = pltpu.get_barrier_semaphore()
        for o in (1, 2, 3):
            pl.semaphore_signal(
                barrier_sem, inc=1,
                device_id=(lax.rem(my + o, N_DEV),),
                device_id_type=pl.DeviceIdType.MESH,
            )
        pl.semaphore_wait(barrier_sem, 3)

        rs_sends = []
        for o in (1, 2, 3):
            peer = lax.rem(my + o, N_DEV)
            rdma = pltpu.make_async_remote_copy(
                src_ref=send_buf.at[o - 1],
                dst_ref=rs_buf.at[3 - o],
                send_sem=rs_send_sems.at[o - 1],
                recv_sem=rs_recv_sems.at[3 - o],
                device_id=(peer,),
                device_id_type=pl.DeviceIdType.MESH,
            )
            rdma.start()
            rs_sends.append(rdma)

        acc = p_ref[pl.ds(my * c, c), :].astype(jnp.float32)
        for slot in range(3):
            recv = pltpu.make_async_remote_copy(
                src_ref=rs_buf.at[slot],
                dst_ref=rs_buf.at[slot],
                send_sem=rs_send_sems.at[slot],
                recv_sem=rs_recv_sems.at[slot],
                device_id=(my,),
                device_id_type=pl.DeviceIdType.MESH,
            )
            recv.wait_recv()
            acc = acc + rs_buf[slot, :, :].astype(jnp.float32)

        out_ref[pl.ds(my * c, c), :] = acc
        red_bf[:, :] = acc.astype(jnp.bfloat16)
        for rdma in rs_sends:
            rdma.wait_send()

        ag_sends = []
        for o in (1, 2, 3):
            peer = lax.rem(my + o, N_DEV)
            rdma = pltpu.make_async_remote_copy(
                src_ref=red_bf,
                dst_ref=ag_buf.at[3 - o],
                send_sem=ag_send_sems.at[o - 1],
                recv_sem=ag_recv_sems.at[3 - o],
                device_id=(peer,),
                device_id_type=pl.DeviceIdType.MESH,
            )
            rdma.start()
            ag_sends.append(rdma)

        for slot in range(3):
            recv = pltpu.make_async_remote_copy(
                src_ref=ag_buf.at[slot],
                dst_ref=ag_buf.at[slot],
                send_sem=ag_send_sems.at[slot],
                recv_sem=ag_recv_sems.at[slot],
                device_id=(my,),
                device_id_type=pl.DeviceIdType.MESH,
            )
            recv.wait_recv()
            peer = lax.rem(my + slot + 1, N_DEV)
            out_ref[pl.ds(peer * c, c), :] = ag_buf[slot, :, :].astype(
                jnp.float32
            )
        for rdma in ag_sends:
            rdma.wait_send()

    return pl.pallas_call(
        body,
        out_shape=jax.ShapeDtypeStruct((m, n), jnp.float32),
        in_specs=[pl.BlockSpec(memory_space=pltpu.VMEM)],
        out_specs=pl.BlockSpec(memory_space=pltpu.VMEM),
        scratch_shapes=[
            pltpu.VMEM((3, c, n), jnp.bfloat16),
            pltpu.VMEM((3, c, n), jnp.bfloat16),
            pltpu.VMEM((3, c, n), jnp.bfloat16),
            pltpu.VMEM((c, n), jnp.bfloat16),
            pltpu.SemaphoreType.DMA((3,)),
            pltpu.SemaphoreType.DMA((3,)),
            pltpu.SemaphoreType.DMA((3,)),
            pltpu.SemaphoreType.DMA((3,)),
        ],
        compiler_params=pltpu.CompilerParams(collective_id=0),
    )(partial_bf)


def kernel(x, Wq, K_ext, V_ext, Wo):
    B, Sq, E = x.shape
    hq = Wq.shape[1] // DH
    Skv = K_ext.shape[1]
    my = lax.axis_index("i")

    K = lax.dynamic_slice_in_dim(K_ext, my * hq, hq, axis=2)
    V = lax.dynamic_slice_in_dim(V_ext, my * hq, hq, axis=2)

    bf = jnp.bfloat16
    Q = (x.astype(bf) @ Wq.astype(bf)).reshape(B, Sq, hq, DH)

    qb = (jnp.arange(Sq) // BLK)[:, None]
    kb = (jnp.arange(Skv) // BLK)[None, :]
    mask = (qb == kb) | (kb == 0) | ((qb + kb) % 3 == 0)

    scores = jnp.einsum(
        "bihd,bjhd->bhij", Q, K.astype(bf),
        preferred_element_type=jnp.float32,
    ) * 0.125
    scores = jnp.where(mask[None, None, :, :], scores, -1e9)
    w = jax.nn.softmax(scores, axis=-1)

    ctx = jnp.einsum(
        "bhij,bjhd->bihd", w.astype(bf), V.astype(bf),
        preferred_element_type=jnp.float32,
    ).reshape(B, Sq, hq * DH)

    partial = jnp.einsum(
        "bse,eo->bso", ctx.astype(bf), Wo.astype(bf),
        preferred_element_type=jnp.float32,
    )

    out = _allreduce_rsag(partial.astype(bf).reshape(B * Sq, E))
    return out.reshape(B, Sq, E)
